# Initial kernel scaffold; baseline (speedup 1.0000x reference)
import jax
import jax.numpy as jnp
from jax.experimental import pallas as pl


def kernel(x, edge_index, W1, W2):
    raise NotImplementedError("write your pallas kernel here")


# speedup vs baseline: 6.1399x; 6.1399x over previous
import functools

import jax
import jax.numpy as jnp
from jax import lax
from jax.experimental import pallas as pl
from jax.experimental.pallas import tpu as pltpu
from jax.experimental.pallas import tpu_sc as plsc

NC = 2
NS = 16
CH = 128


def _make_seg_sum(n, d, e):
    assert e % CH == 0
    nchunks = e // CH
    nworkers = NC * NS
    base_ch = nchunks // nworkers
    rem_ch = nchunks % nworkers
    rpt = (n // NS) & ~7
    tail = n - NS * rpt
    tail_off = NS * rpt
    assert tail % 8 == 0

    mesh = plsc.VectorSubcoreMesh(core_axis_name="c", subcore_axis_name="s",
                                  num_cores=NC, num_subcores=NS)

    @functools.partial(
        pl.kernel,
        out_type=jax.ShapeDtypeStruct((NC, n, d), jnp.float32),
        mesh=mesh,
        scratch_types=[
            pltpu.VMEM((CH,), jnp.int32),
            pltpu.VMEM((CH,), jnp.int32),
            pltpu.VMEM((CH, d), jnp.float32),
            pltpu.VMEM_SHARED((n, d), jnp.float32),
            pltpu.SemaphoreType.DMA,
        ],
    )
    def seg_sum(x_hbm, src_hbm, dst_hbm, zero_hbm, out_hbm,
                src_v, dst_v, rows_v, acc_sh, sem):
        c = lax.axis_index("c")
        s = lax.axis_index("s")
        wid = s * NC + c

        pltpu.sync_copy(zero_hbm.at[pl.ds(s * rpt, rpt)],
                        acc_sh.at[pl.ds(s * rpt, rpt)])
        if tail:
            @pl.when(s == 0)
            def _zero_tail():
                pltpu.sync_copy(zero_hbm.at[pl.ds(tail_off, tail)],
                                acc_sh.at[pl.ds(tail_off, tail)])
        plsc.subcore_barrier()

        nch = base_ch + jnp.where(wid < rem_ch, 1, 0)

        def body(j, carry):
            off = (j * nworkers + wid) * CH
            pltpu.sync_copy(src_hbm.at[pl.ds(off, CH)], src_v)
            pltpu.sync_copy(dst_hbm.at[pl.ds(off, CH)], dst_v)
            pltpu.async_copy(x_hbm.at[src_v], rows_v, sem).wait()
            pltpu.sync_copy(rows_v, acc_sh.at[dst_v], add=True)
            return carry

        lax.fori_loop(0, nch, body, 0)
        plsc.subcore_barrier()

        pltpu.sync_copy(acc_sh.at[pl.ds(s * rpt, rpt)],
                        out_hbm.at[c, pl.ds(s * rpt, rpt)])
        if tail:
            @pl.when(s == 0)
            def _write_tail():
                pltpu.sync_copy(acc_sh.at[pl.ds(tail_off, tail)],
                                out_hbm.at[c, pl.ds(tail_off, tail)])

    return seg_sum


def _mm_body(x_ref, p0_ref, p1_ref, w_ref, o_ref):
    m = x_ref[...] + p0_ref[...] + p1_ref[...]
    o_ref[...] = lax.dot_general(m, w_ref[...], (((1,), (1,)), ((), ())),
                                 preferred_element_type=jnp.float32)


def _fused_combine_matmul(x, p0, p1, w):
    n, d = x.shape
    blk = 1000
    assert n % blk == 0
    return pl.pallas_call(
        _mm_body,
        grid=(n // blk,),
        in_specs=[
            pl.BlockSpec((blk, d), lambda i: (i, 0)),
            pl.BlockSpec((blk, d), lambda i: (i, 0)),
            pl.BlockSpec((blk, d), lambda i: (i, 0)),
            pl.BlockSpec((d, d), lambda i: (0, 0)),
        ],
        out_specs=pl.BlockSpec((blk, d), lambda i: (i, 0)),
        out_shape=jax.ShapeDtypeStruct((n, d), jnp.float32),
    )(x, p0, p1, w)


def kernel(x, edge_index, W1, W2):
    n, d = x.shape
    e = edge_index.shape[1]
    src = edge_index[0]
    dst = edge_index[1]
    zero = jnp.zeros((n, d), jnp.float32)

    seg_sum = _make_seg_sum(n, d, e)
    p = seg_sum(x, src, dst, zero)
    h = _fused_combine_matmul(x, p[0], p[1], W1)
    q = seg_sum(h, src, dst, zero)
    out = _fused_combine_matmul(h, q[0], q[1], W2)
    return out
